# baseline (speedup 1.0000x reference)
import functools

import jax
import jax.numpy as jnp
from jax import lax
from jax.experimental import pallas as pl
from jax.experimental.pallas import tpu as pltpu
from jax.experimental.pallas import tpu_sc as plsc

B = 4096
L = 200
H = 64
NT = 32

_NC = 2
_NS = 16
_NW = _NC * _NS
_BPW = B // _NW


def _pool_body(idx_hbm, table_hbm, out_hbm, idx_v, rows_v, acc_v, sem0, sem1):
    wid = lax.axis_index("s") * _NC + lax.axis_index("c")
    base = wid * _BPW
    pltpu.sync_copy(idx_hbm.at[pl.ds(base, _BPW)], idx_v)

    sems = (sem0, sem1)

    def start(s, buf):
        for hf in range(2):
            pltpu.async_copy(
                table_hbm.at[idx_v.at[s, hf]], rows_v.at[buf, hf], sems[buf])

    def wait(s, buf):
        for hf in range(2):
            pltpu.make_async_copy(
                table_hbm.at[idx_v.at[s, hf]], rows_v.at[buf, hf], sems[buf]
            ).wait()

    def accum(s, buf):
        z = jnp.zeros((16,), jnp.float32)

        def body_for(hf):
            def body(i, c):
                a = list(c)
                for u in range(8):
                    j = i * 8 + u
                    half = (u % 2) * 4
                    for k in range(4):
                        a[half + k] = a[half + k] + rows_v[buf, hf, j,
                                                          pl.ds(16 * k, 16)]
                return tuple(a)
            return body

        a = lax.fori_loop(0, 128 // 8, body_for(0), (z,) * 8)
        a = lax.fori_loop(0, 72 // 8, body_for(1), a)
        for k in range(4):
            acc_v[s, pl.ds(16 * k, 16)] = a[k] + a[4 + k]

    start(0, 0)

    def pair(p, carry):
        s0 = 2 * p
        start(s0 + 1, 1)
        wait(s0, 0)
        accum(s0, 0)

        @pl.when(p < _BPW // 2 - 1)
        def _():
            start(s0 + 2, 0)

        wait(s0 + 1, 1)
        accum(s0 + 1, 1)
        return carry

    lax.fori_loop(0, _BPW // 2, pair, 0)
    pltpu.sync_copy(acc_v, out_hbm.at[pl.ds(base, _BPW)])


@functools.cache
def _pool():
    return pl.kernel(
        _pool_body,
        mesh=plsc.VectorSubcoreMesh(core_axis_name="c", subcore_axis_name="s"),
        compiler_params=pltpu.CompilerParams(use_tc_tiling_on_sc=False),
        out_type=jax.ShapeDtypeStruct((B, H), jnp.float32),
        scratch_types=[
            pltpu.VMEM((_BPW, 2, 128), jnp.int32),
            pltpu.VMEM((2, 2, 128, H), jnp.float32),
            pltpu.VMEM((_BPW, H), jnp.float32),
            pltpu.SemaphoreType.DMA,
            pltpu.SemaphoreType.DMA,
        ],
    )


def _softplus(x):
    return jnp.maximum(x, 0.0) + jnp.log1p(jnp.exp(-jnp.abs(x)))


def _dot_t(x, w):
    return lax.dot_general(x, w, (((1,), (1,)), ((), ())),
                           preferred_element_type=jnp.float32)


def _mlp_body(pooled_ref, rd_ref, W1a_ref, w1b_ref, b1_ref, W2_ref, b2_ref,
              Wmu_ref, bmu_ref, Wlv_ref, blv_ref, gmu_ref, betamu_ref,
              glv_ref, betalv_ref, loc_ref, scale_ref):
    rd = rd_ref[...]
    ave = pooled_ref[...] / rd
    lrd = jnp.log(rd)
    h = _dot_t(ave, W1a_ref[...]) + lrd * w1b_ref[...] + b1_ref[...]
    h = _softplus(h)
    h = _softplus(_dot_t(h, W2_ref[...]) + b2_ref[...])
    tl = _dot_t(h, Wmu_ref[...]) + bmu_ref[...]
    ts = _dot_t(h, Wlv_ref[...]) + blv_ref[...]
    eps = 1e-5
    ml = jnp.mean(tl, axis=0, keepdims=True)
    vl = jnp.mean((tl - ml) * (tl - ml), axis=0, keepdims=True)
    loc_ref[...] = (tl - ml) * lax.rsqrt(vl + eps) * gmu_ref[...] + betamu_ref[...]
    ms = jnp.mean(ts, axis=0, keepdims=True)
    vs = jnp.mean((ts - ms) * (ts - ms), axis=0, keepdims=True)
    scale_ref[...] = jnp.exp(
        0.5 * ((ts - ms) * lax.rsqrt(vs + eps) * glv_ref[...] + betalv_ref[...]))


_mlp = pl.pallas_call(
    _mlp_body,
    out_shape=(
        jax.ShapeDtypeStruct((B, NT), jnp.float32),
        jax.ShapeDtypeStruct((B, NT), jnp.float32),
    ),
)


def kernel(idx, read_depth, table, W1, b1, W2, b2, Wmu, bmu, Wlv, blv,
           gmu, betamu, glv, betalv):
    idx_pad = jnp.concatenate(
        [idx.astype(jnp.int32),
         jnp.zeros((B, 2 * 128 - L), jnp.int32)], axis=1).reshape(B, 2, 128)
    pooled = _pool()(idx_pad, table)
    return _mlp(pooled, read_depth,
                W1[:, :H], W1[:, H][None, :], b1[None, :],
                W2, b2[None, :],
                Wmu, bmu[None, :], Wlv, blv[None, :],
                gmu[None, :], betamu[None, :], glv[None, :], betalv[None, :])


# speedup vs baseline: 1.0005x; 1.0005x over previous
import functools

import jax
import jax.numpy as jnp
from jax import lax
from jax.experimental import pallas as pl
from jax.experimental.pallas import tpu as pltpu
from jax.experimental.pallas import tpu_sc as plsc

B = 4096
L = 200
H = 64
NT = 32

_NC = 2
_NS = 16
_NW = _NC * _NS
_BPW = B // _NW


_DEPTH = 8


def _pool_body(idx_hbm, table_hbm, out_hbm, idx_v, ring_v, acc_v, *sems):
    wid = lax.axis_index("s") * _NC + lax.axis_index("c")
    base = wid * _BPW
    pltpu.sync_copy(idx_hbm.at[pl.ds(base, _BPW)], idx_v)

    def start(u, s, hf):
        pltpu.async_copy(table_hbm.at[idx_v.at[s, hf]], ring_v.at[u], sems[u])

    def wait(u, s, hf):
        pltpu.make_async_copy(
            table_hbm.at[idx_v.at[s, hf]], ring_v.at[u], sems[u]).wait()

    def chunk_accum(u, a_init, n_groups):
        def body(i, c):
            a = list(c)
            for g in range(8):
                j = i * 8 + g
                half = (g % 2) * 4
                for k in range(4):
                    a[half + k] = a[half + k] + ring_v[u, j, pl.ds(16 * k, 16)]
            return tuple(a)
        return lax.fori_loop(0, n_groups, body, a_init)

    for u in range(_DEPTH):
        start(u, u // 2, u % 2)

    zs = (jnp.zeros((16,), jnp.float32),) * 8
    nblk = _BPW // (_DEPTH // 2)

    def block(b, carry):
        a = zs
        for u in range(_DEPTH):
            s = (_DEPTH // 2) * b + u // 2
            wait(u, s, u % 2)
            if u % 2 == 0:
                a = chunk_accum(u, zs, 128 // 8)
            else:
                a = chunk_accum(u, a, 72 // 8)
                for k in range(4):
                    acc_v[s, pl.ds(16 * k, 16)] = a[k] + a[4 + k]

            @pl.when(b < nblk - 1)
            def _():
                start(u, s + _DEPTH // 2, u % 2)
        return carry

    lax.fori_loop(0, nblk, block, 0)
    pltpu.sync_copy(acc_v, out_hbm.at[pl.ds(base, _BPW)])


@functools.cache
def _pool():
    return pl.kernel(
        _pool_body,
        mesh=plsc.VectorSubcoreMesh(core_axis_name="c", subcore_axis_name="s"),
        compiler_params=pltpu.CompilerParams(use_tc_tiling_on_sc=False),
        out_type=jax.ShapeDtypeStruct((B, H), jnp.float32),
        scratch_types=[
            pltpu.VMEM((_BPW, 2, 128), jnp.int32),
            pltpu.VMEM((_DEPTH, 128, H), jnp.float32),
            pltpu.VMEM((_BPW, H), jnp.float32),
        ] + [pltpu.SemaphoreType.DMA] * _DEPTH,
    )


def _softplus(x):
    return jnp.maximum(x, 0.0) + jnp.log1p(jnp.exp(-jnp.abs(x)))


def _dot_t(x, w):
    return lax.dot_general(x, w, (((1,), (1,)), ((), ())),
                           preferred_element_type=jnp.float32)


def _mlp_body(pooled_ref, rd_ref, W1a_ref, w1b_ref, b1_ref, W2_ref, b2_ref,
              Wmu_ref, bmu_ref, Wlv_ref, blv_ref, gmu_ref, betamu_ref,
              glv_ref, betalv_ref, loc_ref, scale_ref):
    rd = rd_ref[...]
    ave = pooled_ref[...] / rd
    lrd = jnp.log(rd)
    h = _dot_t(ave, W1a_ref[...]) + lrd * w1b_ref[...] + b1_ref[...]
    h = _softplus(h)
    h = _softplus(_dot_t(h, W2_ref[...]) + b2_ref[...])
    tl = _dot_t(h, Wmu_ref[...]) + bmu_ref[...]
    ts = _dot_t(h, Wlv_ref[...]) + blv_ref[...]
    eps = 1e-5
    ml = jnp.mean(tl, axis=0, keepdims=True)
    vl = jnp.mean((tl - ml) * (tl - ml), axis=0, keepdims=True)
    loc_ref[...] = (tl - ml) * lax.rsqrt(vl + eps) * gmu_ref[...] + betamu_ref[...]
    ms = jnp.mean(ts, axis=0, keepdims=True)
    vs = jnp.mean((ts - ms) * (ts - ms), axis=0, keepdims=True)
    scale_ref[...] = jnp.exp(
        0.5 * ((ts - ms) * lax.rsqrt(vs + eps) * glv_ref[...] + betalv_ref[...]))


_mlp = pl.pallas_call(
    _mlp_body,
    out_shape=(
        jax.ShapeDtypeStruct((B, NT), jnp.float32),
        jax.ShapeDtypeStruct((B, NT), jnp.float32),
    ),
)


def kernel(idx, read_depth, table, W1, b1, W2, b2, Wmu, bmu, Wlv, blv,
           gmu, betamu, glv, betalv):
    idx_pad = jnp.concatenate(
        [idx.astype(jnp.int32),
         jnp.zeros((B, 2 * 128 - L), jnp.int32)], axis=1).reshape(B, 2, 128)
    pooled = _pool()(idx_pad, table)
    return _mlp(pooled, read_depth,
                W1[:, :H], W1[:, H][None, :], b1[None, :],
                W2, b2[None, :],
                Wmu, bmu[None, :], Wlv, blv[None, :],
                gmu[None, :], betamu[None, :], glv[None, :], betalv[None, :])


# speedup vs baseline: 3.7540x; 3.7523x over previous
import functools

import jax
import jax.numpy as jnp
from jax import lax
from jax.experimental import pallas as pl
from jax.experimental.pallas import tpu as pltpu
from jax.experimental.pallas import tpu_sc as plsc

B = 4096
L = 200
H = 64
NT = 32

_NC = 2
_NS = 16
_NW = _NC * _NS
_BPW = B // _NW


_NG = 13
_LP = _NG * 16


def _pool_body(idx_hbm, table_hbm, out_hbm, idx_v, ring_v, acc_v, *sems):
    wid = lax.axis_index("s") * _NC + lax.axis_index("c")
    base = wid * _BPW
    pltpu.sync_copy(idx_hbm.at[pl.ds(base, _BPW)], idx_v)

    def ivec(s, g):
        return idx_v[s, pl.ds(16 * g, 16)]

    def start(g, iv):
        pltpu.async_copy(table_hbm.at[iv], ring_v.at[g], sems[g])

    def wait(g, iv):
        pltpu.make_async_copy(table_hbm.at[iv], ring_v.at[g], sems[g]).wait()

    def group_accum(g, a, nrows):
        a = list(a)
        for j in range(nrows):
            half = (j % 2) * 4
            for k in range(4):
                a[half + k] = a[half + k] + ring_v[g, j, pl.ds(16 * k, 16)]
        return tuple(a)

    for g in range(_NG):
        start(g, ivec(0, g))

    zs = (jnp.zeros((16,), jnp.float32),) * 8

    def sample(s, carry):
        sn = jnp.minimum(s + 1, _BPW - 1)
        a = zs
        for g in range(_NG):
            iv_next = ivec(sn, g)
            wait(g, iv_next)
            a = group_accum(g, a, 16 if g < _NG - 1 else 8)

            @pl.when(s < _BPW - 1)
            def _():
                start(g, iv_next)
        for k in range(4):
            acc_v[s, pl.ds(16 * k, 16)] = a[k] + a[4 + k]
        return carry

    lax.fori_loop(0, _BPW, sample, 0)
    pltpu.sync_copy(acc_v, out_hbm.at[pl.ds(base, _BPW)])


@functools.cache
def _pool():
    return pl.kernel(
        _pool_body,
        mesh=plsc.VectorSubcoreMesh(core_axis_name="c", subcore_axis_name="s"),
        compiler_params=pltpu.CompilerParams(use_tc_tiling_on_sc=False),
        out_type=jax.ShapeDtypeStruct((B, H), jnp.float32),
        scratch_types=[
            pltpu.VMEM((_BPW, _LP), jnp.int32),
            pltpu.VMEM((_NG, 16, H), jnp.float32),
            pltpu.VMEM((_BPW, H), jnp.float32),
        ] + [pltpu.SemaphoreType.DMA] * _NG,
    )


def _softplus(x):
    return jnp.maximum(x, 0.0) + jnp.log1p(jnp.exp(-jnp.abs(x)))


def _dot_t(x, w):
    return lax.dot_general(x, w, (((1,), (1,)), ((), ())),
                           preferred_element_type=jnp.float32)


def _mlp_body(pooled_ref, rd_ref, W1a_ref, w1b_ref, b1_ref, W2_ref, b2_ref,
              Wmu_ref, bmu_ref, Wlv_ref, blv_ref, gmu_ref, betamu_ref,
              glv_ref, betalv_ref, loc_ref, scale_ref):
    rd = rd_ref[...]
    ave = pooled_ref[...] / rd
    lrd = jnp.log(rd)
    h = _dot_t(ave, W1a_ref[...]) + lrd * w1b_ref[...] + b1_ref[...]
    h = _softplus(h)
    h = _softplus(_dot_t(h, W2_ref[...]) + b2_ref[...])
    tl = _dot_t(h, Wmu_ref[...]) + bmu_ref[...]
    ts = _dot_t(h, Wlv_ref[...]) + blv_ref[...]
    eps = 1e-5
    ml = jnp.mean(tl, axis=0, keepdims=True)
    vl = jnp.mean((tl - ml) * (tl - ml), axis=0, keepdims=True)
    loc_ref[...] = (tl - ml) * lax.rsqrt(vl + eps) * gmu_ref[...] + betamu_ref[...]
    ms = jnp.mean(ts, axis=0, keepdims=True)
    vs = jnp.mean((ts - ms) * (ts - ms), axis=0, keepdims=True)
    scale_ref[...] = jnp.exp(
        0.5 * ((ts - ms) * lax.rsqrt(vs + eps) * glv_ref[...] + betalv_ref[...]))


_mlp = pl.pallas_call(
    _mlp_body,
    out_shape=(
        jax.ShapeDtypeStruct((B, NT), jnp.float32),
        jax.ShapeDtypeStruct((B, NT), jnp.float32),
    ),
)


def kernel(idx, read_depth, table, W1, b1, W2, b2, Wmu, bmu, Wlv, blv,
           gmu, betamu, glv, betalv):
    idx_pad = jnp.concatenate(
        [idx.astype(jnp.int32), jnp.zeros((B, _LP - L), jnp.int32)], axis=1)
    pooled = _pool()(idx_pad, table)
    return _mlp(pooled, read_depth,
                W1[:, :H], W1[:, H][None, :], b1[None, :],
                W2, b2[None, :],
                Wmu, bmu[None, :], Wlv, blv[None, :],
                gmu[None, :], betamu[None, :], glv[None, :], betalv[None, :])
